# baseline (speedup 1.0000x reference)
import functools

import jax
import jax.numpy as jnp
from jax import lax
from jax.experimental import pallas as pl
from jax.experimental.pallas import tpu as pltpu
from jax.experimental.pallas import tpu_sc as plsc

_N = 10000
_E = 320000
_NC = 2
_NS = 16

_EP = 327680


@functools.lru_cache(maxsize=None)
def _make_message_pass(Dh, C, IB, NB, EDGES=_E):
  EPT = EDGES // _NS
  CPT = EPT // C
  BLKS = CPT // IB
  RNDS = IB // NB
  IBC = IB * C
  assert CPT % IB == 0 and BLKS % 2 == 0 and IB % NB == 0
  assert C % 8 == 0 and IBC % 16 == 0
  mesh = plsc.VectorSubcoreMesh(
      core_axis_name="c", subcore_axis_name="s",
      num_cores=_NC, num_subcores=_NS)
  rows_pt = _N // _NS

  @functools.partial(
      pl.kernel,
      out_type=(jax.ShapeDtypeStruct((_N, Dh), jnp.float32),
                jax.ShapeDtypeStruct((_N, Dh), jnp.float32)),
      mesh=mesh,
      scratch_types=[
          pltpu.VMEM((2 * IBC,), jnp.int32),
          pltpu.VMEM((2 * IB, C), jnp.int32),
          pltpu.VMEM((2 * IBC,), jnp.float32),
          [pltpu.VMEM((C, Dh), jnp.float32)] * NB,
          [pltpu.SemaphoreType.DMA] * NB,
          [pltpu.SemaphoreType.DMA] * NB,
          [pltpu.SemaphoreType.DMA] * 2,
          pltpu.VMEM_SHARED((_N, Dh), jnp.float32),
      ],
      compiler_params=pltpu.CompilerParams(
          use_tc_tiling_on_sc=False, needs_layout_passes=False),
  )
  def mp(tabA, tabB, srcf, dst2, wf, initA, initB, outA, outB,
         srcb, dstb, wb, rows, gsem, ssem, isem, accum):
    c = lax.axis_index("c")
    s = lax.axis_index("s")
    r0 = s * rows_pt
    ebase = s * EPT
    row_base = s * CPT

    def idx_start(slot, blk):
      fsl = pl.ds(slot * IBC, IBC)
      pltpu.async_copy(srcf.at[pl.ds(ebase + blk * IBC, IBC)],
                       srcb.at[fsl], isem[slot])
      pltpu.async_copy(dst2.at[pl.ds(row_base + blk * IB, IB)],
                       dstb.at[pl.ds(slot * IB, IB)], isem[slot])
      pltpu.async_copy(wf.at[pl.ds(ebase + blk * IBC, IBC)],
                       wb.at[fsl], isem[slot])

    def idx_wait(slot):
      fsl = pl.ds(slot * IBC, IBC)
      pltpu.make_async_copy(srcf.at[pl.ds(0, IBC)], srcb.at[fsl],
                            isem[slot]).wait()
      pltpu.make_async_copy(dst2.at[pl.ds(0, IB)],
                            dstb.at[pl.ds(slot * IB, IB)], isem[slot]).wait()
      pltpu.make_async_copy(wf.at[pl.ds(0, IBC)], wb.at[fsl],
                            isem[slot]).wait()

    def gather_start(b, slot, lch):
      idx = srcb.at[pl.ds(slot * IBC + lch * C, C)]

      @pl.when(c == 0)
      def _():
        pltpu.async_copy(tabA.at[idx], rows[b], gsem[b])

      @pl.when(c == 1)
      def _():
        pltpu.async_copy(tabB.at[idx], rows[b], gsem[b])

    def gather_wait(b):
      pltpu.make_async_copy(tabA.at[srcb.at[pl.ds(0, C)]], rows[b],
                            gsem[b]).wait()

    def scatter_start(b, slot, lch):
      pltpu.async_copy(rows[b], accum.at[dstb.at[slot * IB + lch]],
                       ssem[b], add=True)

    def scatter_wait(b):
      pltpu.make_async_copy(rows[b], accum.at[dstb.at[0]], ssem[b]).wait()

    def scale(b, slot, lch):
      wbase = slot * IBC + lch * C

      @plsc.parallel_loop(0, C, 1, unroll=8)
      def _(i):
        wv = plsc.load_gather(
            wb, [jnp.broadcast_to(wbase + i, (16,)).astype(jnp.int32)])
        r = rows[b]
        for d in range(Dh // 16):
          sl = pl.ds(d * 16, 16)
          r[i, sl] = r[i, sl] * wv

    idx_start(0, 0)

    @pl.when(c == 0)
    def _():
      pltpu.sync_copy(initA.at[pl.ds(r0, rows_pt)],
                      accum.at[pl.ds(r0, rows_pt)])

    @pl.when(c == 1)
    def _():
      pltpu.sync_copy(initB.at[pl.ds(r0, rows_pt)],
                      accum.at[pl.ds(r0, rows_pt)])

    plsc.subcore_barrier()

    def blk_pair_body(p, _):
      for slot in (0, 1):
        blk = p * 2 + slot
        idx_wait(slot)

        @pl.when(blk + 1 < BLKS)
        def _():
          idx_start(1 - slot, blk + 1)

        gather_start(0, slot, 0)
        gather_start(1, slot, 1)
        gather_start(2, slot, 2)

        def round_body(t, _):
          for b in range(NB):
            lch = t * NB + b
            fb = (b + 3) % NB
            if b >= NB - 3:
              scatter_wait(fb)
            else:
              @pl.when(t > 0)
              def _():
                scatter_wait(fb)

            @pl.when(lch + 3 < IB)
            def _():
              gather_start(fb, slot, lch + 3)

            gather_wait(b)
            scale(b, slot, lch)
            scatter_start(b, slot, lch)
          return 0
        lax.fori_loop(0, RNDS, round_body, 0)

        for k in range(NB - 3):
          scatter_wait((k + 3) % NB)
      return 0
    lax.fori_loop(0, BLKS // 2, blk_pair_body, 0)

    plsc.subcore_barrier()

    @pl.when(c == 0)
    def _():
      pltpu.sync_copy(accum.at[pl.ds(r0, rows_pt)],
                      outA.at[pl.ds(r0, rows_pt)])

    @pl.when(c == 1)
    def _():
      pltpu.sync_copy(accum.at[pl.ds(r0, rows_pt)],
                      outB.at[pl.ds(r0, rows_pt)])

  return mp


_BM = 2000
_HI = jax.lax.Precision.HIGHEST


def _dense01(aggA, aggB, x, Wc0, b0, Wc1, b1):
  def body(aggA_r, aggB_r, x_r, Wc0_r, b0_r, Wc1_r, b1_r,
           y2a, y2b, r2a, r2b):
    cc = jnp.concatenate([aggA_r[...], aggB_r[...], x_r[...]], axis=1)
    h1 = jnp.maximum(jnp.dot(cc, Wc0_r[...], precision=_HI) + b0_r[...], 0.0)
    yr = jnp.dot(h1, Wc1_r[...], precision=_HI)
    y2a[...] = yr[:, 0:128]
    y2b[...] = yr[:, 128:256]
    r2a[...] = yr[:, 256:384] + b1_r[:, 0:128]
    r2b[...] = yr[:, 384:512] + b1_r[:, 128:256]

  full = lambda shape: pl.BlockSpec(shape, lambda i: (0,) * len(shape))
  row = lambda d: pl.BlockSpec((_BM, d), lambda i: (i, 0))
  return pl.pallas_call(
      body,
      grid=(_N // _BM,),
      in_specs=[row(64), row(64), row(128),
                full((256, 256)), full((1, 256)), full((256, 512)),
                full((1, 256))],
      out_specs=[row(128)] * 4,
      out_shape=[jax.ShapeDtypeStruct((_N, 128), jnp.float32)] * 4,
  )(aggA, aggB, x, Wc0, b0.reshape(1, 256), Wc1, b1.reshape(1, 256))


def _dense2(h2a, h2b, Wc2, b2):
  def body(ha, hb, Wc2_r, b2_r, y3a, y3b, r3a, r3b):
    cc = jnp.concatenate(
        [jnp.maximum(ha[...], 0.0), jnp.maximum(hb[...], 0.0)], axis=1)
    yr = jnp.dot(cc, Wc2_r[...], precision=_HI)
    y3a[...] = yr[:, 0:64]
    y3b[...] = yr[:, 64:128]
    r3a[...] = yr[:, 128:192] + b2_r[:, 0:64]
    r3b[...] = yr[:, 192:256] + b2_r[:, 64:128]

  full = lambda shape: pl.BlockSpec(shape, lambda i: (0,) * len(shape))
  row = lambda d: pl.BlockSpec((_BM, d), lambda i: (i, 0))
  return pl.pallas_call(
      body,
      grid=(_N // _BM,),
      in_specs=[row(128), row(128), full((256, 256)), full((1, 128))],
      out_specs=[row(64)] * 4,
      out_shape=[jax.ShapeDtypeStruct((_N, 64), jnp.float32)] * 4,
  )(h2a, h2b, Wc2, b2.reshape(1, 128))


def kernel(x, edge_index, edge_weight, W_rel0, b_rel0, W_root0,
           W_rel1, b_rel1, W_root1, W_rel2, b_rel2, W_root2):
  pad = _EP - _E
  src = edge_index[0].astype(jnp.int32)
  dst = edge_index[1].astype(jnp.int32)
  w = edge_weight
  srcp = jnp.pad(src, (0, pad))
  dstp = jnp.pad(dst, (0, pad))
  wp = jnp.pad(w, (0, pad))
  dst80u = dst.reshape(_E // 80, 80)
  dst80p = dstp.reshape(_EP // 80, 80)
  zeros64 = jnp.zeros((_N, 64), jnp.float32)

  aggA, aggB = _make_message_pass(64, 80, 25, 5)(
      x[:, :64], x[:, 64:], src, dst80u, w, zeros64, zeros64)

  Wc0 = jnp.concatenate([W_rel0, W_root0], axis=0)
  Wc1 = jnp.concatenate([W_rel1, W_root1], axis=1)
  y2a, y2b, r2a, r2b = _dense01(aggA, aggB, x, Wc0, b_rel0, Wc1, b_rel1)
  o1a, o1b = _make_message_pass(128, 40, 50, 5)(
      y2a, y2b, src, dst.reshape(_E // 40, 40), w, r2a, r2b)

  Wc2 = jnp.concatenate([W_rel2, W_root2], axis=1)
  y3a, y3b, r3a, r3b = _dense2(o1a, o1b, Wc2, b_rel2)
  outA, outB = _make_message_pass(64, 80, 25, 5)(
      y3a, y3b, src, dst80u, w, r3a, r3b)

  return jnp.concatenate([outA, outB], axis=1)


# speedup vs baseline: 1.0097x; 1.0097x over previous
import functools

import jax
import jax.numpy as jnp
from jax import lax
from jax.experimental import pallas as pl
from jax.experimental.pallas import tpu as pltpu
from jax.experimental.pallas import tpu_sc as plsc

_N = 10000
_E = 320000
_NC = 2
_NS = 16

_EP = 327680


@functools.lru_cache(maxsize=None)
def _make_message_pass(Dh, C, IB, NB, EDGES=_E):
  EPT = EDGES // _NS
  CPT = EPT // C
  BLKS = CPT // IB
  RNDS = IB // NB
  IBC = IB * C
  assert CPT % IB == 0 and BLKS % 2 == 0 and IB % NB == 0
  assert C % 8 == 0 and IBC % 16 == 0
  mesh = plsc.VectorSubcoreMesh(
      core_axis_name="c", subcore_axis_name="s",
      num_cores=_NC, num_subcores=_NS)
  rows_pt = _N // _NS

  @functools.partial(
      pl.kernel,
      out_type=(jax.ShapeDtypeStruct((_N, Dh), jnp.float32),
                jax.ShapeDtypeStruct((_N, Dh), jnp.float32)),
      mesh=mesh,
      scratch_types=[
          pltpu.VMEM((2 * IBC,), jnp.int32),
          pltpu.VMEM((2 * IB, C), jnp.int32),
          pltpu.VMEM((2 * IBC,), jnp.float32),
          [pltpu.VMEM((C, Dh), jnp.float32)] * NB,
          [pltpu.SemaphoreType.DMA] * NB,
          [pltpu.SemaphoreType.DMA] * NB,
          [pltpu.SemaphoreType.DMA] * 2,
          pltpu.VMEM_SHARED((_N, Dh), jnp.float32),
      ],
      compiler_params=pltpu.CompilerParams(
          use_tc_tiling_on_sc=False, needs_layout_passes=False),
  )
  def mp(tabA, tabB, srcf, dst2, wf, initA, initB, outA, outB,
         srcb, dstb, wb, rows, gsem, ssem, isem, accum):
    c = lax.axis_index("c")
    s = lax.axis_index("s")
    r0 = s * rows_pt
    ebase = s * EPT
    row_base = s * CPT

    def idx_start(slot, blk):
      fsl = pl.ds(slot * IBC, IBC)
      pltpu.async_copy(srcf.at[pl.ds(ebase + blk * IBC, IBC)],
                       srcb.at[fsl], isem[slot])
      pltpu.async_copy(dst2.at[pl.ds(row_base + blk * IB, IB)],
                       dstb.at[pl.ds(slot * IB, IB)], isem[slot])
      pltpu.async_copy(wf.at[pl.ds(ebase + blk * IBC, IBC)],
                       wb.at[fsl], isem[slot])

    def idx_wait(slot):
      fsl = pl.ds(slot * IBC, IBC)
      pltpu.make_async_copy(srcf.at[pl.ds(0, IBC)], srcb.at[fsl],
                            isem[slot]).wait()
      pltpu.make_async_copy(dst2.at[pl.ds(0, IB)],
                            dstb.at[pl.ds(slot * IB, IB)], isem[slot]).wait()
      pltpu.make_async_copy(wf.at[pl.ds(0, IBC)], wb.at[fsl],
                            isem[slot]).wait()

    def gather_start(b, slot, lch):
      idx = srcb.at[pl.ds(slot * IBC + lch * C, C)]

      @pl.when(c == 0)
      def _():
        pltpu.async_copy(tabA.at[idx], rows[b], gsem[b])

      @pl.when(c == 1)
      def _():
        pltpu.async_copy(tabB.at[idx], rows[b], gsem[b])

    def gather_wait(b):
      pltpu.make_async_copy(tabA.at[srcb.at[pl.ds(0, C)]], rows[b],
                            gsem[b]).wait()

    def scatter_start(b, slot, lch):
      pltpu.async_copy(rows[b], accum.at[dstb.at[slot * IB + lch]],
                       ssem[b], add=True)

    def scatter_wait(b):
      pltpu.make_async_copy(rows[b], accum.at[dstb.at[0]], ssem[b]).wait()

    def scale(b, slot, lch):
      wbase = slot * IBC + lch * C

      @plsc.parallel_loop(0, C, 1, unroll=4)
      def _(i):
        wv = plsc.load_gather(
            wb, [jnp.broadcast_to(wbase + i, (16,)).astype(jnp.int32)])
        r = rows[b]
        for d in range(Dh // 16):
          sl = pl.ds(d * 16, 16)
          r[i, sl] = r[i, sl] * wv

    idx_start(0, 0)

    @pl.when(c == 0)
    def _():
      pltpu.sync_copy(initA.at[pl.ds(r0, rows_pt)],
                      accum.at[pl.ds(r0, rows_pt)])

    @pl.when(c == 1)
    def _():
      pltpu.sync_copy(initB.at[pl.ds(r0, rows_pt)],
                      accum.at[pl.ds(r0, rows_pt)])

    plsc.subcore_barrier()

    def blk_pair_body(p, _):
      for slot in (0, 1):
        blk = p * 2 + slot
        idx_wait(slot)

        @pl.when(blk + 1 < BLKS)
        def _():
          idx_start(1 - slot, blk + 1)

        gather_start(0, slot, 0)
        gather_start(1, slot, 1)
        gather_start(2, slot, 2)

        def round_body(t, _):
          for b in range(NB):
            lch = t * NB + b
            fb = (b + 3) % NB
            if b >= NB - 3:
              scatter_wait(fb)
            else:
              @pl.when(t > 0)
              def _():
                scatter_wait(fb)

            @pl.when(lch + 3 < IB)
            def _():
              gather_start(fb, slot, lch + 3)

            gather_wait(b)
            scale(b, slot, lch)
            scatter_start(b, slot, lch)
          return 0
        lax.fori_loop(0, RNDS, round_body, 0)

        for k in range(NB - 3):
          scatter_wait((k + 3) % NB)
      return 0
    lax.fori_loop(0, BLKS // 2, blk_pair_body, 0)

    plsc.subcore_barrier()

    @pl.when(c == 0)
    def _():
      pltpu.sync_copy(accum.at[pl.ds(r0, rows_pt)],
                      outA.at[pl.ds(r0, rows_pt)])

    @pl.when(c == 1)
    def _():
      pltpu.sync_copy(accum.at[pl.ds(r0, rows_pt)],
                      outB.at[pl.ds(r0, rows_pt)])

  return mp


_BM = 2000
_HI = jax.lax.Precision.HIGHEST


def _dense01(aggA, aggB, x, Wc0, b0, Wc1, b1):
  def body(aggA_r, aggB_r, x_r, Wc0_r, b0_r, Wc1_r, b1_r,
           y2a, y2b, r2a, r2b):
    cc = jnp.concatenate([aggA_r[...], aggB_r[...], x_r[...]], axis=1)
    h1 = jnp.maximum(jnp.dot(cc, Wc0_r[...], precision=_HI) + b0_r[...], 0.0)
    yr = jnp.dot(h1, Wc1_r[...], precision=_HI)
    y2a[...] = yr[:, 0:128]
    y2b[...] = yr[:, 128:256]
    r2a[...] = yr[:, 256:384] + b1_r[:, 0:128]
    r2b[...] = yr[:, 384:512] + b1_r[:, 128:256]

  full = lambda shape: pl.BlockSpec(shape, lambda i: (0,) * len(shape))
  row = lambda d: pl.BlockSpec((_BM, d), lambda i: (i, 0))
  return pl.pallas_call(
      body,
      grid=(_N // _BM,),
      in_specs=[row(64), row(64), row(128),
                full((256, 256)), full((1, 256)), full((256, 512)),
                full((1, 256))],
      out_specs=[row(128)] * 4,
      out_shape=[jax.ShapeDtypeStruct((_N, 128), jnp.float32)] * 4,
  )(aggA, aggB, x, Wc0, b0.reshape(1, 256), Wc1, b1.reshape(1, 256))


def _dense2(h2a, h2b, Wc2, b2):
  def body(ha, hb, Wc2_r, b2_r, y3a, y3b, r3a, r3b):
    cc = jnp.concatenate(
        [jnp.maximum(ha[...], 0.0), jnp.maximum(hb[...], 0.0)], axis=1)
    yr = jnp.dot(cc, Wc2_r[...], precision=_HI)
    y3a[...] = yr[:, 0:64]
    y3b[...] = yr[:, 64:128]
    r3a[...] = yr[:, 128:192] + b2_r[:, 0:64]
    r3b[...] = yr[:, 192:256] + b2_r[:, 64:128]

  full = lambda shape: pl.BlockSpec(shape, lambda i: (0,) * len(shape))
  row = lambda d: pl.BlockSpec((_BM, d), lambda i: (i, 0))
  return pl.pallas_call(
      body,
      grid=(_N // _BM,),
      in_specs=[row(128), row(128), full((256, 256)), full((1, 128))],
      out_specs=[row(64)] * 4,
      out_shape=[jax.ShapeDtypeStruct((_N, 64), jnp.float32)] * 4,
  )(h2a, h2b, Wc2, b2.reshape(1, 128))


def kernel(x, edge_index, edge_weight, W_rel0, b_rel0, W_root0,
           W_rel1, b_rel1, W_root1, W_rel2, b_rel2, W_root2):
  pad = _EP - _E
  src = edge_index[0].astype(jnp.int32)
  dst = edge_index[1].astype(jnp.int32)
  w = edge_weight
  srcp = jnp.pad(src, (0, pad))
  dstp = jnp.pad(dst, (0, pad))
  wp = jnp.pad(w, (0, pad))
  dst80u = dst.reshape(_E // 80, 80)
  dst80p = dstp.reshape(_EP // 80, 80)
  zeros64 = jnp.zeros((_N, 64), jnp.float32)

  aggA, aggB = _make_message_pass(64, 80, 25, 5)(
      x[:, :64], x[:, 64:], src, dst80u, w, zeros64, zeros64)

  Wc0 = jnp.concatenate([W_rel0, W_root0], axis=0)
  Wc1 = jnp.concatenate([W_rel1, W_root1], axis=1)
  y2a, y2b, r2a, r2b = _dense01(aggA, aggB, x, Wc0, b_rel0, Wc1, b_rel1)
  o1a, o1b = _make_message_pass(128, 40, 50, 5)(
      y2a, y2b, src, dst.reshape(_E // 40, 40), w, r2a, r2b)

  Wc2 = jnp.concatenate([W_rel2, W_root2], axis=1)
  y3a, y3b, r3a, r3b = _dense2(o1a, o1b, Wc2, b_rel2)
  outA, outB = _make_message_pass(64, 80, 25, 5)(
      y3a, y3b, src, dst80u, w, r3a, r3b)

  return jnp.concatenate([outA, outB], axis=1)
